# baseline (speedup 1.0000x reference)
import functools

import jax
import jax.numpy as jnp
from jax import lax
from jax.experimental import pallas as pl
from jax.experimental.pallas import tpu as pltpu
from jax.experimental.pallas import tpu_sc as plsc

B = 16384
D = 64
N_ENT = 1000000
N_REL = 1000
L = 16
NC = 2
NS = 16
NW = NC * NS
BPW = B // NW
C = 128
NCHUNK = BPW // C
NGRP = C // L


def _sc_body(h_hbm, t_hbm, r_hbm, ent1_hbm, ent2_hbm, rel1_hbm, rel2_hbm,
             res_hbm,
             hv, tv, rv, ih, it, ir, e1h, e2h, e1t, e2t, r1c, r2c, resc, sem):
    wid = lax.axis_index("s") * NC + lax.axis_index("c")

    for chunk in range(NCHUNK):
        base = wid * BPW + chunk * C
        pltpu.sync_copy(h_hbm.at[pl.ds(base, C)], hv)
        pltpu.sync_copy(t_hbm.at[pl.ds(base, C)], tv)
        pltpu.sync_copy(r_hbm.at[pl.ds(base, C)], rv)

        def idx_body(kv_ref, out_ref, stride):
            kvs = [kv_ref[pl.ds(g * L, L)] for g in range(NGRP)]

            def d_body(d, _):
                off = d * stride
                for g in range(NGRP):
                    out_ref[pl.ds(d * C + g * L, L)] = kvs[g] + off
                return 0

            lax.fori_loop(0, D, d_body, 0)

        idx_body(hv, ih, N_ENT)
        idx_body(tv, it, N_ENT)
        idx_body(rv, ir, N_REL)

        cps = [
            pltpu.async_copy(ent1_hbm.at[ih], e1h, sem),
            pltpu.async_copy(ent2_hbm.at[ih], e2h, sem),
            pltpu.async_copy(ent1_hbm.at[it], e1t, sem),
            pltpu.async_copy(ent2_hbm.at[it], e2t, sem),
            pltpu.async_copy(rel1_hbm.at[ir], r1c, sem),
            pltpu.async_copy(rel2_hbm.at[ir], r2c, sem),
        ]
        for cp in cps:
            cp.wait()

        def d_body(d, accs):
            out = []
            for g in range(NGRP):
                sl = pl.ds(d * C + g * L, L)
                a1 = e1h[sl]
                a2 = e2h[sl]
                b1 = e1t[sl]
                b2 = e2t[sl]
                q1 = r1c[sl]
                q2 = r2c[sl]
                out.append(accs[g] + q1 * (a1 * b1 + a2 * b2)
                           + q2 * (a1 * b2 - a2 * b1))
            return tuple(out)

        accs = lax.fori_loop(
            0, D, d_body, tuple(jnp.zeros((L,), jnp.float32) for _ in range(NGRP)))
        for g in range(NGRP):
            resc[pl.ds(g * L, L)] = accs[g]
        pltpu.sync_copy(resc, res_hbm.at[pl.ds(base, C)])


def _make_sc_kernel():
    mesh = plsc.VectorSubcoreMesh(core_axis_name="c", subcore_axis_name="s")
    return pl.kernel(
        _sc_body,
        out_type=jax.ShapeDtypeStruct((B,), jnp.float32),
        mesh=mesh,
        compiler_params=pltpu.CompilerParams(
            needs_layout_passes=False, use_tc_tiling_on_sc=False),
        scratch_types=[
            pltpu.VMEM((C,), jnp.int32),
            pltpu.VMEM((C,), jnp.int32),
            pltpu.VMEM((C,), jnp.int32),
            pltpu.VMEM((D * C,), jnp.int32),
            pltpu.VMEM((D * C,), jnp.int32),
            pltpu.VMEM((D * C,), jnp.int32),
            pltpu.VMEM((D * C,), jnp.float32),
            pltpu.VMEM((D * C,), jnp.float32),
            pltpu.VMEM((D * C,), jnp.float32),
            pltpu.VMEM((D * C,), jnp.float32),
            pltpu.VMEM((D * C,), jnp.float32),
            pltpu.VMEM((D * C,), jnp.float32),
            pltpu.VMEM((C,), jnp.float32),
            pltpu.SemaphoreType.DMA,
        ],
    )


def _loss_body(res_ref, y_ref, out_ref):
    x = -y_ref[...] * res_ref[...]
    out_ref[0, 0] = jnp.mean(jax.nn.softplus(x))


@jax.jit
def kernel(h, t, r, y, ent1, ent2, rel1, rel2):
    h = h.astype(jnp.int32)
    t = t.astype(jnp.int32)
    r = r.astype(jnp.int32)
    ent1l = ent1.T.reshape(N_ENT * D)
    ent2l = ent2.T.reshape(N_ENT * D)
    rel1l = rel1.T.reshape(N_REL * D)
    rel2l = rel2.T.reshape(N_REL * D)
    res = _make_sc_kernel()(h, t, r, ent1l, ent2l, rel1l, rel2l)
    loss = pl.pallas_call(
        _loss_body,
        out_shape=jax.ShapeDtypeStruct((1, 1), jnp.float32),
        out_specs=pl.BlockSpec(memory_space=pltpu.SMEM),
    )(res.reshape(128, 128), y.reshape(128, 128))
    return loss[0, 0]


# speedup vs baseline: 9.0511x; 9.0511x over previous
import jax
import jax.numpy as jnp
from jax import lax
from jax.experimental import pallas as pl
from jax.experimental.pallas import tpu as pltpu
from jax.experimental.pallas import tpu_sc as plsc

B = 16384
D = 64
L = 16
NC = 2
NS = 16
NW = NC * NS
BPW = B // NW
C = 128
NCHUNK = BPW // C
NGRP = D // L


def _sc_body(h_hbm, t_hbm, r_hbm, ent1_hbm, ent2_hbm, rel1_hbm, rel2_hbm,
             out_hbm,
             hv, tv, rv, e1h, e2h, e1t, e2t, r1c, r2c, resc, sem):
    wid = lax.axis_index("s") * NC + lax.axis_index("c")

    for chunk in range(NCHUNK):
        base = wid * BPW + chunk * C
        pltpu.sync_copy(h_hbm.at[pl.ds(base, C)], hv)
        pltpu.sync_copy(t_hbm.at[pl.ds(base, C)], tv)
        pltpu.sync_copy(r_hbm.at[pl.ds(base, C)], rv)

        cp1 = pltpu.make_async_copy(ent1_hbm.at[hv], e1h, sem)
        cp2 = pltpu.make_async_copy(ent2_hbm.at[hv], e2h, sem)
        cp3 = pltpu.make_async_copy(ent1_hbm.at[tv], e1t, sem)
        cp4 = pltpu.make_async_copy(ent2_hbm.at[tv], e2t, sem)
        cp5 = pltpu.make_async_copy(rel1_hbm.at[rv], r1c, sem)
        cp6 = pltpu.make_async_copy(rel2_hbm.at[rv], r2c, sem)
        for cp in (cp1, cp2, cp3, cp4, cp5, cp6):
            cp.start()
        for cp in (cp1, cp2, cp3, cp4, cp5, cp6):
            cp.wait()

        def e_body(e, carry):
            acc = jnp.zeros((L,), jnp.float32)
            for g in range(NGRP):
                sl = pl.ds(g * L, L)
                a1 = e1h[e, sl]
                a2 = e2h[e, sl]
                b1 = e1t[e, sl]
                b2 = e2t[e, sl]
                q1 = r1c[e, sl]
                q2 = r2c[e, sl]
                acc = acc + q1 * (a1 * b1 + a2 * b2) + q2 * (a1 * b2 - a2 * b1)
            resc[e, :] = acc
            return carry

        lax.fori_loop(0, C, e_body, 0)
        pltpu.sync_copy(resc, out_hbm.at[pl.ds(base, C)])


def _make_sc_kernel():
    mesh = plsc.VectorSubcoreMesh(core_axis_name="c", subcore_axis_name="s")
    return pl.kernel(
        _sc_body,
        out_type=jax.ShapeDtypeStruct((B, L), jnp.float32),
        mesh=mesh,
        compiler_params=pltpu.CompilerParams(use_tc_tiling_on_sc=False),
        scratch_types=[
            pltpu.VMEM((C,), jnp.int32),
            pltpu.VMEM((C,), jnp.int32),
            pltpu.VMEM((C,), jnp.int32),
            pltpu.VMEM((C, D), jnp.float32),
            pltpu.VMEM((C, D), jnp.float32),
            pltpu.VMEM((C, D), jnp.float32),
            pltpu.VMEM((C, D), jnp.float32),
            pltpu.VMEM((C, D), jnp.float32),
            pltpu.VMEM((C, D), jnp.float32),
            pltpu.VMEM((C, L), jnp.float32),
            pltpu.SemaphoreType.DMA,
        ],
    )


def _loss_body(res_ref, y_ref, out_ref):
    s = jnp.sum(res_ref[...], axis=2)
    out_ref[0, 0] = jnp.mean(jax.nn.softplus(-y_ref[...] * s))


@jax.jit
def kernel(h, t, r, y, ent1, ent2, rel1, rel2):
    h = h.astype(jnp.int32)
    t = t.astype(jnp.int32)
    r = r.astype(jnp.int32)
    partial = _make_sc_kernel()(h, t, r, ent1, ent2, rel1, rel2)
    loss = pl.pallas_call(
        _loss_body,
        out_shape=jax.ShapeDtypeStruct((1, 1), jnp.float32),
        out_specs=pl.BlockSpec(memory_space=pltpu.SMEM),
    )(partial.reshape(128, 128, L), y.reshape(128, 128))
    return loss[0, 0]
